# baseline (speedup 1.0000x reference)
import functools

import jax
import jax.numpy as jnp
from jax import lax
from jax.experimental import pallas as pl
from jax.experimental.pallas import tpu as pltpu
from jax.experimental.pallas import tpu_sc as plsc

_NC = 2
_NS = 16
_NW = _NC * _NS
_L = 16


@functools.lru_cache(maxsize=None)
def _make_gather(V, D, J, B):
    assert B % (_NW * 128) == 0 and D % 8 == 0
    ntc = B // 128
    tc_per_w = ntc // _NW
    dt = D // 8
    mesh = plsc.VectorSubcoreMesh(core_axis_name="c", subcore_axis_name="s")

    @functools.partial(
        pl.kernel,
        mesh=mesh,
        compiler_params=pltpu.CompilerParams(
            use_tc_tiling_on_sc=False, needs_layout_passes=False
        ),
        out_type=jax.ShapeDtypeStruct((J, dt, ntc, 8, 128), jnp.float32),
        scratch_types=[
            pltpu.VMEM((128,), jnp.int32),
            pltpu.VMEM((128, D), jnp.float32),
            pltpu.VMEM((dt, 8, 128), jnp.float32),
            pltpu.SemaphoreType.DMA,
            pltpu.SemaphoreType.DMA,
            pltpu.SemaphoreType.DMA,
        ],
    )
    def gather_kernel(table, idx, out, idx_v, rows_v, perm_v, sem_i, sem_g, sem_o):
        w = lax.axis_index("s") * _NC + lax.axis_index("c")
        tc = w
        lane = lax.iota(jnp.int32, _L)

        def j_body(j, _):
            pltpu.async_copy(idx.at[pl.ds(j * B + tc * 128, 128)], idx_v, sem_i).wait()
            pltpu.async_copy(table.at[idx_v], rows_v, sem_g).wait()
            for d in range(D):
                for l0 in range(0, 128, _L):
                    vals = plsc.load_gather(
                        rows_v, [l0 + lane, jnp.full((_L,), d, jnp.int32)]
                    )
                    perm_v[d // 8, d % 8, pl.ds(l0, _L)] = vals
            pltpu.async_copy(perm_v, out.at[j, :, tc], sem_o).wait()
            return 0

        lax.fori_loop(0, J, j_body, 0)

    return gather_kernel


def kernel(x, weight):
    Bq, J = x.shape
    V, D = weight.shape
    idx_lin = x.T.reshape(J * Bq).astype(jnp.int32)
    out5 = _make_gather(V, D, J, Bq)(weight, idx_lin)
    return out5.transpose(2, 4, 0, 1, 3).reshape(Bq, J, D)
